# Initial kernel scaffold; baseline (speedup 1.0000x reference)
import jax
import jax.numpy as jnp
from jax.experimental import pallas as pl


def kernel(memory, usage_count, importance_score, temporal_index, encoded_experiences, next_index_start):
    raise NotImplementedError("write your pallas kernel here")


# speedup vs baseline: 24.8514x; 24.8514x over previous
import functools

import jax
import jax.numpy as jnp
from jax import lax
from jax.experimental import pallas as pl
from jax.experimental.pallas import tpu as pltpu
from jax.experimental.pallas import tpu_sc as plsc

CAP = 100000
DIM = 256
BATCH = 1024
L = 16
NT = 16
BLK = 256
NB = 392
NPAD = NB * BLK
MPAD = 400
CHUNK = 2000
CROWS = 32
NCORES = 2
CORE_ROWS = CAP // NCORES
NCH = 2 * ((CORE_ROWS // 15) // (2 * CROWS))
ROWS_PER_TILE = NCH * CROWS
TAIL_ROWS = CORE_ROWS - 15 * ROWS_PER_TILE
TAIL_CH = TAIL_ROWS // CROWS
TAIL_LEFT = TAIL_ROWS - TAIL_CH * CROWS

_MESH = plsc.VectorSubcoreMesh(
    core_axis_name="c", subcore_axis_name="s", num_cores=NCORES,
    num_subcores=NT)


@functools.partial(
    pl.kernel,
    out_type=jax.ShapeDtypeStruct((CAP, DIM), jnp.float32),
    mesh=_MESH,
    scratch_types=[
        pltpu.VMEM((NPAD,), jnp.float32),
        pltpu.VMEM((MPAD,), jnp.float32),
        pltpu.VMEM((2 * 16,), jnp.float32),
        pltpu.VMEM((BATCH,), jnp.float32),
        pltpu.VMEM((BATCH,), jnp.int32),
        pltpu.VMEM((CHUNK,), jnp.float32),
        pltpu.VMEM((CHUNK,), jnp.float32),
        pltpu.VMEM((32, DIM), jnp.float32),
        pltpu.VMEM((32, DIM), jnp.float32),
        pltpu.VMEM((32,), jnp.int32),
        pltpu.VMEM((32,), jnp.int32),
        pltpu.VMEM((64,), jnp.int32),
        pltpu.VMEM((80,), jnp.int32),
        pltpu.VMEM((80,), jnp.int32),
        pltpu.VMEM_SHARED((BATCH,), jnp.int32),
        pltpu.SemaphoreType.DMA,
        pltpu.SemaphoreType.DMA,
        pltpu.SemaphoreType.DMA,
        pltpu.SemaphoreType.DMA,
        pltpu.SemaphoreType.DMA,
    ],
    compiler_params=pltpu.CompilerParams(needs_layout_passes=False),
)
def _sc_kernel(mem_h, uc_h, imp_h, exp_h, out_h,
               S, M, T2, nrm, sel_v, ucb, ipb, ebuf, cbuf, gidx, sidx, frv,
               cslot, crow, sel_sh, sem, sema, semb, semra, semrb):
    wid = lax.axis_index("s")
    cid = lax.axis_index("c")
    lo = cid * CORE_ROWS
    iota = lax.iota(jnp.int32, L)
    INF = jnp.float32(jnp.inf)
    BIG = jnp.int32(1 << 30)

    def load1(ref, idx, zero):
        off = (idx // L) * L
        lane = idx - off
        v = ref[pl.ds(off, L)]
        return jnp.sum(jnp.where(iota == lane, v, zero))

    def store1(ref, idx, val):
        off = (idx // L) * L
        lane = idx - off
        vv = ref[pl.ds(off, L)]
        ref[pl.ds(off, L)] = jnp.where(iota == lane, val, vv)

    @pl.when(wid == 0)
    def _tile0():
        def nchunk(c, carry):
            pltpu.sync_copy(exp_h.at[pl.ds(c * 32, 32)], ebuf)

            def nrow(r, carry2):
                acc = ebuf[r, pl.ds(0, L)] * ebuf[r, pl.ds(0, L)]
                for k in range(1, DIM // L):
                    v = ebuf[r, pl.ds(k * L, L)]
                    acc = acc + v * v
                store1(nrm, c * 32 + r, jnp.sum(acc))
                return carry2

            return lax.fori_loop(0, 32, nrow, carry)

        lax.fori_loop(0, BATCH // 32, nchunk, 0)

        def nsq(v, carry):
            x = nrm[pl.ds(v * L, L)]
            b = plsc.bitcast(x, jnp.int32)
            y = plsc.bitcast((b >> 1) + jnp.int32(0x1FBD1DF5), jnp.float32)
            for _ in range(4):
                y = jnp.float32(0.5) * (y + x / y)
            nrm[pl.ds(v * L, L)] = y
            return carry

        lax.fori_loop(0, BATCH // L, nsq, 0)

        def ptail(v, carry):
            S[pl.ds(CAP + v * L, L)] = jnp.full((L,), INF, jnp.float32)
            return carry

        lax.fori_loop(0, (NPAD - CAP) // L, ptail, 0)

        def schunk(c, carry):
            pltpu.sync_copy(uc_h.at[pl.ds(c * CHUNK, CHUNK)], ucb)
            pltpu.sync_copy(imp_h.at[pl.ds(c * CHUNK, CHUNK)], ipb)

            def sv(v, carry2):
                S[pl.ds(c * CHUNK + v * L, L)] = (
                    ucb[pl.ds(v * L, L)] * ipb[pl.ds(v * L, L)])
                return carry2

            return lax.fori_loop(0, CHUNK // L, sv, carry)

        lax.fori_loop(0, CAP // CHUNK, schunk, 0)

        def mb(bidx, carry):
            mn = S[pl.ds(bidx * BLK, L)]
            for k in range(1, BLK // L):
                mn = jnp.minimum(mn, S[pl.ds(bidx * BLK + k * L, L)])
            store1(M, bidx, jnp.min(mn))
            return carry

        M[pl.ds(MPAD - L, L)] = jnp.full((L,), INF, jnp.float32)
        lax.fori_loop(0, NB, mb, 0)

        T2[pl.ds(L, L)] = jnp.full((L,), INF, jnp.float32)

        def t2b(j, carry):
            store1(T2, j, jnp.min(M[pl.ds(j * L, L)]))
            return carry

        lax.fori_loop(0, MPAD // L, t2b, 0)

        def step(i, carry):
            t0 = T2[pl.ds(0, L)]
            t1 = T2[pl.ds(L, L)]
            m = jnp.min(jnp.minimum(t0, t1))
            cand = jnp.where(t0 == m, iota,
                             jnp.where(t1 == m, iota + L, BIG))
            g = jnp.min(cand)
            v1 = M[pl.ds(g * L, L)]
            bstar = g * L + plsc.all_reduce_ffs(v1 == m)[0]
            base = bstar * BLK

            candv = jnp.full((L,), BIG, jnp.int32)
            for k in range(BLK // L):
                vk = S[pl.ds(base + k * L, L)]
                candv = jnp.minimum(
                    candv, jnp.where(vk == m, iota + k * L, BIG))
            r = base + jnp.min(candv)
            store1(sel_v, i, r)

            nv = load1(nrm, i, jnp.float32(0.0))
            off = (r // L) * L
            lane = r - off
            vv = S[pl.ds(off, L)]
            S[pl.ds(off, L)] = jnp.where(iota == lane, nv, vv)

            mn = S[pl.ds(base, L)]
            for k in range(1, BLK // L):
                mn = jnp.minimum(mn, S[pl.ds(base + k * L, L)])
            store1(M, bstar, jnp.min(mn))
            store1(T2, g, jnp.min(M[pl.ds(g * L, L)]))
            return carry

        lax.fori_loop(0, BATCH, step, 0)
        pltpu.sync_copy(sel_v, sel_sh)

    @pl.when(wid > 0)
    def _copy():
        start = lo + (wid - 1) * ROWS_PER_TILE
        pltpu.async_copy(mem_h.at[pl.ds(start, CROWS)], ebuf, semra)
        pltpu.async_copy(mem_h.at[pl.ds(start + CROWS, CROWS)], cbuf, semrb)

        def cpair(p, carry):
            c0 = start + (2 * p) * CROWS
            c1 = c0 + CROWS
            pltpu.make_async_copy(
                mem_h.at[pl.ds(c0, CROWS)], ebuf, semra).wait()
            pltpu.async_copy(ebuf, out_h.at[pl.ds(c0, CROWS)], sema)
            pltpu.make_async_copy(
                mem_h.at[pl.ds(c1, CROWS)], cbuf, semrb).wait()
            pltpu.async_copy(cbuf, out_h.at[pl.ds(c1, CROWS)], semb)

            @pl.when(p < NCH // 2 - 1)
            def _():
                pltpu.make_async_copy(
                    ebuf, out_h.at[pl.ds(c0, CROWS)], sema).wait()
                pltpu.async_copy(
                    mem_h.at[pl.ds(c0 + 2 * CROWS, CROWS)], ebuf, semra)
                pltpu.make_async_copy(
                    cbuf, out_h.at[pl.ds(c1, CROWS)], semb).wait()
                pltpu.async_copy(
                    mem_h.at[pl.ds(c1 + 2 * CROWS, CROWS)], cbuf, semrb)

            return carry

        lax.fori_loop(0, NCH // 2, cpair, 0)
        e0 = start + (NCH - 2) * CROWS
        pltpu.make_async_copy(ebuf, out_h.at[pl.ds(e0, CROWS)], sema).wait()
        pltpu.make_async_copy(
            cbuf, out_h.at[pl.ds(e0 + CROWS, CROWS)], semb).wait()

    @pl.when(wid == NT - 1)
    def _copy_tail():
        t0 = lo + 15 * ROWS_PER_TILE

        def tbody(c, carry):
            pltpu.sync_copy(mem_h.at[pl.ds(t0 + c * CROWS, CROWS)], ebuf)
            pltpu.sync_copy(ebuf, out_h.at[pl.ds(t0 + c * CROWS, CROWS)])
            return carry

        lax.fori_loop(0, TAIL_CH, tbody, 0)
        if TAIL_LEFT:
            t1 = t0 + TAIL_CH * CROWS
            pltpu.sync_copy(mem_h.at[pl.ds(t1, TAIL_LEFT)],
                            ebuf.at[pl.ds(0, TAIL_LEFT)])
            pltpu.sync_copy(ebuf.at[pl.ds(0, TAIL_LEFT)],
                            out_h.at[pl.ds(t1, TAIL_LEFT)])

    plsc.subcore_barrier()

    pltpu.sync_copy(sel_sh, sel_v)

    def dstep(t, carry):
        i = wid * 64 + t
        slot = load1(sel_v, i, jnp.int32(0))
        runm = jnp.full((L,), jnp.int32(-1))
        for j in range(BATCH // L):
            v = sel_v[pl.ds(j * L, L)]
            cand = jnp.where(v == slot, iota + j * L, jnp.int32(-1))
            runm = jnp.maximum(runm, cand)
        store1(frv, t, jnp.max(runm))
        return carry

    lax.fori_loop(0, 64, dstep, 0)

    def cstep(t, cnt):
        i = wid * 64 + t
        slot = load1(sel_v, i, jnp.int32(0))
        row = load1(frv, t, jnp.int32(0))
        owned = (slot >= lo) & (slot < lo + CORE_ROWS)

        @pl.when(owned)
        def _():
            store1(cslot, cnt, slot)
            store1(crow, cnt, row)

        return jnp.where(owned, cnt + 1, cnt)

    total = lax.fori_loop(0, 64, cstep, jnp.int32(0))

    s0 = cslot[pl.ds(0, L)][0]
    r0 = crow[pl.ds(0, L)][0]
    for v in range(4):
        idxs = iota + v * L
        cs = cslot[pl.ds(v * L, L)]
        cr = crow[pl.ds(v * L, L)]
        cslot[pl.ds(v * L, L)] = jnp.where(idxs < total, cs, s0)
        crow[pl.ds(v * L, L)] = jnp.where(idxs < total, cr, r0)

    def sch(c, carry):
        for v in range(2):
            gidx[pl.ds(v * L, L)] = crow[pl.ds(c * 32 + v * L, L)]
            sidx[pl.ds(v * L, L)] = cslot[pl.ds(c * 32 + v * L, L)]
        pltpu.async_copy(exp_h.at[gidx], ebuf, sem).wait()
        pltpu.async_copy(ebuf, out_h.at[sidx], sem).wait()
        return carry

    lax.fori_loop(0, (total + 31) // 32, sch, 0)


def kernel(memory, usage_count, importance_score, temporal_index,
           encoded_experiences, next_index_start):
    del temporal_index, next_index_start
    return _sc_kernel(memory, usage_count, importance_score,
                      encoded_experiences)
